# baseline (speedup 1.0000x reference)
import functools

import jax
import jax.numpy as jnp
from jax import lax
from jax.experimental import pallas as pl
from jax.experimental.pallas import tpu as pltpu
from jax.experimental.pallas import tpu_sc as plsc

MAXLEN = 8000
HEAD_DIM = 64
SEQ = 2048
N = SEQ * SEQ

NUM_CORES = 2
NUM_SUBCORES = 16
NUM_WORKERS = NUM_CORES * NUM_SUBCORES

IDX_MINOR = 128
ROWS_PER_STEP = 2
CHUNK = ROWS_PER_STEP * IDX_MINOR
NBUF = 4
IDX_ROWS = N // IDX_MINOR
ROWS_PER_WORKER = IDX_ROWS // NUM_WORKERS
STEPS = ROWS_PER_WORKER // ROWS_PER_STEP
GROUPS = STEPS // NBUF
PER_WORKER_OUT = N // NUM_WORKERS


def _sc_body(idx_hbm, table_hbm, out_hbm, idx_v, rows_v, gsems, wsems):
    wid = lax.axis_index("s") * NUM_CORES + lax.axis_index("c")
    row_base = wid * ROWS_PER_WORKER
    out_base = wid * PER_WORKER_OUT

    def out_slice(i):
        flat = out_base + i * CHUNK
        return out_hbm.at[flat // SEQ, pl.ds(flat % SEQ, CHUNK)]

    def stage_and_gather(i, b):
        pltpu.sync_copy(
            idx_hbm.at[pl.ds(row_base + i * ROWS_PER_STEP, ROWS_PER_STEP)],
            idx_v.at[b])
        for j in range(ROWS_PER_STEP):
            for k in range(IDX_MINOR // 16):
                v = idx_v[b, j, pl.ds(k * 16, 16)]
                v = jnp.minimum(jnp.maximum(v, -MAXLEN), MAXLEN - 1) + MAXLEN
                idx_v[b, j, pl.ds(k * 16, 16)] = v
        return [
            pltpu.async_copy(table_hbm.at[idx_v.at[b, j]],
                             rows_v.at[b, pl.ds(j * IDX_MINOR, IDX_MINOR)],
                             gsems[b])
            for j in range(ROWS_PER_STEP)
        ]

    def drain_write(i, b):
        pltpu.make_async_copy(rows_v.at[b], out_slice(i), wsems[b]).wait()

    def group(g, carry):
        gathers = []
        for b in range(NBUF):
            i = g * NBUF + b
            @pl.when(g > 0)
            def _():
                drain_write(i, b)
            gathers.append(stage_and_gather(i, b))
        for b in range(NBUF):
            i = g * NBUF + b
            for cp in gathers[b]:
                cp.wait()
            pltpu.async_copy(rows_v.at[b], out_slice(i), wsems[b])
        return carry

    lax.fori_loop(0, GROUPS, group, 0)
    for b in range(NBUF):
        drain_write((GROUPS - 1) * NBUF + b, b)


@functools.partial(
    pl.kernel,
    out_type=jax.ShapeDtypeStruct((SEQ, SEQ, HEAD_DIM), jnp.float32),
    mesh=plsc.VectorSubcoreMesh(core_axis_name="c", subcore_axis_name="s"),
    scratch_types=[
        pltpu.VMEM((NBUF, ROWS_PER_STEP, IDX_MINOR), jnp.int32),
        pltpu.VMEM((NBUF, CHUNK, HEAD_DIM), jnp.float32),
        [pltpu.SemaphoreType.DMA] * NBUF,
        [pltpu.SemaphoreType.DMA] * NBUF,
    ],
    compiler_params=pltpu.CompilerParams(use_tc_tiling_on_sc=False),
)
def _sc_gather(idx_hbm, table_hbm, out_hbm, idx_v, rows_v, gsems, wsems):
    _sc_body(idx_hbm, table_hbm, out_hbm, idx_v, rows_v, gsems, wsems)


def kernel(pos_seq, pe_k):
    idx2d = pos_seq.astype(jnp.int32).reshape(IDX_ROWS, IDX_MINOR)
    return _sc_gather(idx2d, pe_k)


# speedup vs baseline: 1.3513x; 1.3513x over previous
import functools

import jax
import jax.numpy as jnp
from jax import lax
from jax.experimental import pallas as pl
from jax.experimental.pallas import tpu as pltpu
from jax.experimental.pallas import tpu_sc as plsc

MAXLEN = 8000
HEAD_DIM = 64
SEQ = 2048
N = SEQ * SEQ

NUM_CORES = 2
NUM_SUBCORES = 16
NUM_WORKERS = NUM_CORES * NUM_SUBCORES

IDX_MINOR = 128
ROWS_PER_STEP = 2
CHUNK = ROWS_PER_STEP * IDX_MINOR
NBUF = 2
IDX_ROWS = N // IDX_MINOR
ROWS_PER_WORKER = IDX_ROWS // NUM_WORKERS
STEPS = ROWS_PER_WORKER // ROWS_PER_STEP
GROUPS = STEPS // NBUF
PER_WORKER_OUT = N // NUM_WORKERS


def _sc_body(idx_hbm, table_hbm, out_hbm, idx_v, rows_v, gsems, wsems):
    wid = lax.axis_index("s") * NUM_CORES + lax.axis_index("c")
    row_base = wid * ROWS_PER_WORKER
    out_base = wid * PER_WORKER_OUT

    def out_slice(i):
        flat = out_base + i * CHUNK
        return out_hbm.at[flat // SEQ, pl.ds(flat % SEQ, CHUNK)]

    def stage_and_gather(i, b):
        pltpu.sync_copy(
            idx_hbm.at[pl.ds(row_base + i * ROWS_PER_STEP, ROWS_PER_STEP)],
            idx_v.at[b])
        for j in range(ROWS_PER_STEP):
            for k in range(IDX_MINOR // 16):
                v = idx_v[b, j, pl.ds(k * 16, 16)]
                v = jnp.minimum(jnp.maximum(v, -MAXLEN), MAXLEN - 1) + MAXLEN
                idx_v[b, j, pl.ds(k * 16, 16)] = v
        return [
            pltpu.async_copy(table_hbm.at[idx_v.at[b, j]],
                             rows_v.at[b, pl.ds(j * IDX_MINOR, IDX_MINOR)],
                             gsems[b])
            for j in range(ROWS_PER_STEP)
        ]

    def issue_write(i, b):
        pltpu.async_copy(rows_v.at[b], out_slice(i), wsems[b])

    def drain_write(i, b):
        pltpu.make_async_copy(rows_v.at[b], out_slice(i), wsems[b]).wait()

    def group(g, carry):
        gathers = []
        for b in range(NBUF):
            i = g * NBUF + b
            @pl.when(g > 0)
            def _():
                drain_write(i, b)
            gathers.append(stage_and_gather(i, b))
        for b in range(NBUF):
            i = g * NBUF + b
            for cp in gathers[b]:
                cp.wait()
            issue_write(i, b)
        return carry

    lax.fori_loop(0, GROUPS, group, 0)
    for b in range(NBUF):
        drain_write((GROUPS - 1) * NBUF + b, b)


@functools.partial(
    pl.kernel,
    out_type=jax.ShapeDtypeStruct((SEQ, SEQ, 2 * HEAD_DIM), jnp.float32),
    mesh=plsc.VectorSubcoreMesh(core_axis_name="c", subcore_axis_name="s"),
    scratch_types=[
        pltpu.VMEM((NBUF, ROWS_PER_STEP, IDX_MINOR), jnp.int32),
        pltpu.VMEM((NBUF, CHUNK, 2 * HEAD_DIM), jnp.float32),
        [pltpu.SemaphoreType.DMA] * NBUF,
        [pltpu.SemaphoreType.DMA] * NBUF,
    ],
    compiler_params=pltpu.CompilerParams(use_tc_tiling_on_sc=True),
)
def _sc_gather(idx_hbm, table_hbm, out_hbm, idx_v, rows_v, gsems, wsems):
    _sc_body(idx_hbm, table_hbm, out_hbm, idx_v, rows_v, gsems, wsems)


def kernel(pos_seq, pe_k):
    idx2d = pos_seq.astype(jnp.int32).reshape(IDX_ROWS, IDX_MINOR)
    table = jnp.pad(pe_k, ((0, 0), (0, HEAD_DIM)))
    return _sc_gather(idx2d, table)[:, :, :HEAD_DIM]
